# baseline (speedup 1.0000x reference)
import functools

import jax
import jax.numpy as jnp
from jax import lax
from jax.experimental import pallas as pl
from jax.experimental.pallas import tpu as pltpu
from jax.experimental.pallas import tpu_sc as plsc

_B = 16384
_D = 32
_EPS = 1e-5
_V = 1000001
_CH = 128


_VBLK = 16384
_QSZ = _VBLK // 4
_GSTEPS = (_V + _VBLK - 1) // _VBLK
_MROWS = _GSTEPS * _QSZ


def _repack_body(*refs):
    ins = refs[:4]
    outs = refs[4:]
    eye = jnp.eye(_D, dtype=jnp.float32)
    dn = (((0,), (0,)), ((), ()))
    for t_ref, o_ref in zip(ins, outs):
        qs = [lax.dot_general(t_ref[:, _QSZ * q:_QSZ * (q + 1)], eye, dn,
                              preferred_element_type=jnp.float32)
              for q in range(4)]
        o_ref[...] = jnp.concatenate(qs, axis=1)


def _repack(tabs_t):
    return pl.pallas_call(
        _repack_body,
        grid=(_GSTEPS,),
        in_specs=[pl.BlockSpec((_D, _VBLK), lambda i: (0, i))] * 4,
        out_specs=[pl.BlockSpec((_QSZ, 128), lambda i: (i, 0))] * 4,
        out_shape=[jax.ShapeDtypeStruct((_MROWS, 128), jnp.float32)] * 4,
        compiler_params=pltpu.CompilerParams(
            vmem_limit_bytes=100 * 1024 * 1024),
    )(*tabs_t)


def _sc_gather4(user, item, p_ug, p_ig, p_um, p_im):
    info = plsc.get_sparse_core_info()
    nc, ns = info.num_cores, info.num_subcores
    nw = nc * ns
    bpw = _B // nw

    mesh = plsc.VectorSubcoreMesh(core_axis_name="c", subcore_axis_name="s")

    @functools.partial(
        pl.kernel,
        mesh=mesh,
        out_type=[jax.ShapeDtypeStruct((_B, _D), jnp.float32)] * 4,
        scratch_types=[
            pltpu.VMEM((bpw,), jnp.int32),
            pltpu.VMEM((bpw,), jnp.int32),
            pltpu.VMEM((_CH,), jnp.int32),
            pltpu.VMEM((_CH, 128), jnp.float32),
            pltpu.VMEM((_CH, _D), jnp.float32),
            pltpu.SemaphoreType.DMA,
        ],
        compiler_params=pltpu.CompilerParams(
            use_tc_tiling_on_sc=True, needs_layout_passes=False),
    )
    def k2(user_hbm, item_hbm, p0, p1, p2, p3,
           o0, o1, o2, o3, uidx, iidx, midx, gbuf, sbuf, sem):
        wid = lax.axis_index("s") * nc + lax.axis_index("c")
        base = wid * bpw
        pltpu.sync_copy(user_hbm.at[pl.ds(base, bpw)], uidx)
        pltpu.sync_copy(item_hbm.at[pl.ds(base, bpw)], iidx)
        iota16 = lax.iota(jnp.int32, 16)

        def do_table(packed, out, idx_ref):
            def chunk_body(ch, carry):
                ch_off = pl.multiple_of(ch * _CH, _CH)

                def mids_body(g, carry2):
                    goff = pl.multiple_of(g * 16, 16)
                    v = idx_ref[pl.ds(ch_off + goff, 16)]
                    midx[pl.ds(goff, 16)] = ((v >> 14) << 12) + (v & 4095)
                    return carry2

                lax.fori_loop(0, _CH // 16, mids_body, 0)
                pltpu.async_copy(packed.at[midx], gbuf, sem).wait()

                def sel_body(g, carry2):
                    goff = pl.multiple_of(g * 16, 16)
                    v = idx_ref[pl.ds(ch_off + goff, 16)]
                    colbase = ((v >> 12) & 3) * 32
                    rows = iota16 + goff
                    for f in range(_D):
                        vals = plsc.load_gather(gbuf, [rows, colbase + f])
                        plsc.store_scatter(
                            sbuf, [rows, jnp.zeros((16,), jnp.int32) + f], vals)
                    return carry2

                lax.fori_loop(0, _CH // 16, sel_body, 0)
                pltpu.sync_copy(sbuf, out.at[pl.ds(base + ch_off, _CH)])
                return carry

            lax.fori_loop(0, bpw // _CH, chunk_body, 0)

        do_table(p0, o0, uidx)
        do_table(p1, o1, iidx)
        do_table(p2, o2, uidx)
        do_table(p3, o3, iidx)

    return k2(user, item, p_ug, p_ig, p_um, p_im)


def _tc_body(ug, ig, um, im,
             w0, b0, g0, be0, w1, b1, g1, be1, w2, b2, g2, be2,
             wo_g, wo_m, bo, out):
    x = jnp.concatenate([um[...], im[...]], axis=1)
    for w, b, g, be in ((w0, b0, g0, be0), (w1, b1, g1, be1), (w2, b2, g2, be2)):
        x = jnp.dot(x, w[...], preferred_element_type=jnp.float32) + b[...]
        mean = jnp.mean(x, axis=0, keepdims=True)
        var = jnp.mean((x - mean) ** 2, axis=0, keepdims=True)
        x = (x - mean) * lax.rsqrt(var + _EPS) * g[...] + be[...]
        x = jnp.maximum(x, 0.0)
    gmf = ug[...] * ig[...]
    s = (jnp.dot(gmf, wo_g[...], preferred_element_type=jnp.float32)
         + jnp.dot(x, wo_m[...], preferred_element_type=jnp.float32)
         + bo[...])
    out[...] = 1.0 / (1.0 + jnp.exp(-s))


def _tc_forward(ug, ig, um, im, params):
    return pl.pallas_call(
        _tc_body,
        out_shape=jax.ShapeDtypeStruct((_B, 1), jnp.float32),
        compiler_params=pltpu.CompilerParams(
            vmem_limit_bytes=100 * 1024 * 1024),
    )(ug, ig, um, im, *params)


def kernel(user, item, user_gmf_tab, item_gmf_tab, user_mlp_tab, item_mlp_tab,
           W0, b0, g0, be0, W1, b1, g1, be1, W2, b2, g2, be2, Wo, bo):
    user = user.astype(jnp.int32)
    item = item.astype(jnp.int32)
    packed = _repack((user_gmf_tab.T, item_gmf_tab.T,
                      user_mlp_tab.T, item_mlp_tab.T))
    ug, ig, um, im = _sc_gather4(user, item, *packed)
    params = (
        W0, b0.reshape(1, -1), g0.reshape(1, -1), be0.reshape(1, -1),
        W1, b1.reshape(1, -1), g1.reshape(1, -1), be1.reshape(1, -1),
        W2, b2.reshape(1, -1), g2.reshape(1, -1), be2.reshape(1, -1),
        Wo[:_D], Wo[_D:], bo.reshape(1, 1),
    )
    out = _tc_forward(ug, ig, um, im, params)
    return jnp.squeeze(out, axis=-1)


# speedup vs baseline: 1.0193x; 1.0193x over previous
import functools

import jax
import jax.numpy as jnp
from jax import lax
from jax.experimental import pallas as pl
from jax.experimental.pallas import tpu as pltpu
from jax.experimental.pallas import tpu_sc as plsc

_B = 16384
_D = 32
_EPS = 1e-5
_V = 1000001
_CH = 128


_VBLK = 16384
_QSZ = _VBLK // 4
_GSTEPS = (_V + _VBLK - 1) // _VBLK
_MROWS = _GSTEPS * _QSZ


def _repack_body(*refs):
    ins = refs[:4]
    outs = refs[4:]
    eye = jnp.eye(_D, dtype=jnp.float32)
    dn = (((0,), (0,)), ((), ()))
    for t_ref, o_ref in zip(ins, outs):
        qs = [lax.dot_general(t_ref[:, _QSZ * q:_QSZ * (q + 1)], eye, dn,
                              preferred_element_type=jnp.float32)
              for q in range(4)]
        o_ref[...] = jnp.concatenate(qs, axis=1)


def _repack(tabs_t):
    return pl.pallas_call(
        _repack_body,
        grid=(_GSTEPS,),
        in_specs=[pl.BlockSpec((_D, _VBLK), lambda i: (0, i))] * 4,
        out_specs=[pl.BlockSpec((_QSZ, 128), lambda i: (i, 0))] * 4,
        out_shape=[jax.ShapeDtypeStruct((_MROWS, 128), jnp.float32)] * 4,
        compiler_params=pltpu.CompilerParams(
            vmem_limit_bytes=100 * 1024 * 1024),
    )(*tabs_t)


def _sc_gather4(user, item, p_ug, p_ig, p_um, p_im):
    info = plsc.get_sparse_core_info()
    nc, ns = info.num_cores, info.num_subcores
    nw = nc * ns
    bpw = _B // nw

    mesh = plsc.VectorSubcoreMesh(core_axis_name="c", subcore_axis_name="s")

    @functools.partial(
        pl.kernel,
        mesh=mesh,
        out_type=[jax.ShapeDtypeStruct((_B, 128), jnp.float32)] * 4,
        scratch_types=[
            pltpu.VMEM((bpw,), jnp.int32),
            pltpu.VMEM((bpw,), jnp.int32),
            pltpu.VMEM((_CH,), jnp.int32),
            pltpu.VMEM((_CH, 128), jnp.float32),
            pltpu.SemaphoreType.DMA,
        ],
        compiler_params=pltpu.CompilerParams(
            use_tc_tiling_on_sc=True, needs_layout_passes=False),
    )
    def k2(user_hbm, item_hbm, p0, p1, p2, p3,
           o0, o1, o2, o3, uidx, iidx, midx, gbuf, sem):
        wid = lax.axis_index("s") * nc + lax.axis_index("c")
        base = wid * bpw
        pltpu.sync_copy(user_hbm.at[pl.ds(base, bpw)], uidx)
        pltpu.sync_copy(item_hbm.at[pl.ds(base, bpw)], iidx)

        def do_table(packed, out, idx_ref):
            def chunk_body(ch, carry):
                ch_off = pl.multiple_of(ch * _CH, _CH)

                def mids_body(g, carry2):
                    goff = pl.multiple_of(g * 16, 16)
                    v = idx_ref[pl.ds(ch_off + goff, 16)]
                    midx[pl.ds(goff, 16)] = ((v >> 14) << 12) + (v & 4095)
                    return carry2

                lax.fori_loop(0, _CH // 16, mids_body, 0)
                pltpu.async_copy(packed.at[midx], gbuf, sem).wait()
                pltpu.sync_copy(gbuf, out.at[pl.ds(base + ch_off, _CH)])
                return carry

            lax.fori_loop(0, bpw // _CH, chunk_body, 0)

        do_table(p0, o0, uidx)
        do_table(p1, o1, iidx)
        do_table(p2, o2, uidx)
        do_table(p3, o3, iidx)

    return k2(user, item, p_ug, p_ig, p_um, p_im)


_SBLK = 2048


def _sel_body(uq_ref, iq_ref, x0, x1, x2, x3, o0, o1, o2, o3):
    qs = (uq_ref[...], iq_ref[...], uq_ref[...], iq_ref[...])
    for x_ref, o_ref, q in zip((x0, x1, x2, x3), (o0, o1, o2, o3), qs):
        x = x_ref[...]
        y = jnp.where(q == 0, x[:, 0:32], 0.0)
        for j in range(1, 4):
            y = y + jnp.where(q == j, x[:, 32 * j:32 * (j + 1)], 0.0)
        o_ref[...] = y


def _lane_select(uq, iq, gathered):
    return pl.pallas_call(
        _sel_body,
        grid=(_B // _SBLK,),
        in_specs=[pl.BlockSpec((_SBLK, 1), lambda i: (i, 0))] * 2
        + [pl.BlockSpec((_SBLK, 128), lambda i: (i, 0))] * 4,
        out_specs=[pl.BlockSpec((_SBLK, _D), lambda i: (i, 0))] * 4,
        out_shape=[jax.ShapeDtypeStruct((_B, _D), jnp.float32)] * 4,
    )(uq, iq, *gathered)


def _tc_body(ug, ig, um, im,
             w0, b0, g0, be0, w1, b1, g1, be1, w2, b2, g2, be2,
             wo_g, wo_m, bo, out):
    x = jnp.concatenate([um[...], im[...]], axis=1)
    for w, b, g, be in ((w0, b0, g0, be0), (w1, b1, g1, be1), (w2, b2, g2, be2)):
        x = jnp.dot(x, w[...], preferred_element_type=jnp.float32) + b[...]
        mean = jnp.mean(x, axis=0, keepdims=True)
        var = jnp.mean((x - mean) ** 2, axis=0, keepdims=True)
        x = (x - mean) * lax.rsqrt(var + _EPS) * g[...] + be[...]
        x = jnp.maximum(x, 0.0)
    gmf = ug[...] * ig[...]
    s = (jnp.dot(gmf, wo_g[...], preferred_element_type=jnp.float32)
         + jnp.dot(x, wo_m[...], preferred_element_type=jnp.float32)
         + bo[...])
    out[...] = 1.0 / (1.0 + jnp.exp(-s))


def _tc_forward(ug, ig, um, im, params):
    return pl.pallas_call(
        _tc_body,
        out_shape=jax.ShapeDtypeStruct((_B, 1), jnp.float32),
        compiler_params=pltpu.CompilerParams(
            vmem_limit_bytes=100 * 1024 * 1024),
    )(ug, ig, um, im, *params)


def kernel(user, item, user_gmf_tab, item_gmf_tab, user_mlp_tab, item_mlp_tab,
           W0, b0, g0, be0, W1, b1, g1, be1, W2, b2, g2, be2, Wo, bo):
    user = user.astype(jnp.int32)
    item = item.astype(jnp.int32)
    packed = _repack((user_gmf_tab.T, item_gmf_tab.T,
                      user_mlp_tab.T, item_mlp_tab.T))
    ug, ig, um, im = _sc_gather4(user, item, *packed)
    params = (
        W0, b0.reshape(1, -1), g0.reshape(1, -1), be0.reshape(1, -1),
        W1, b1.reshape(1, -1), g1.reshape(1, -1), be1.reshape(1, -1),
        W2, b2.reshape(1, -1), g2.reshape(1, -1), be2.reshape(1, -1),
        Wo[:_D], Wo[_D:], bo.reshape(1, 1),
    )
    uq = ((user >> 12) & 3).reshape(_B, 1)
    iq = ((item >> 12) & 3).reshape(_B, 1)
    ug, ig, um, im = _lane_select(uq, iq, (ug, ig, um, im))
    out = _tc_forward(ug, ig, um, im, params)
    return jnp.squeeze(out, axis=-1)
